# Initial kernel scaffold; baseline (speedup 1.0000x reference)
import jax
import jax.numpy as jnp
from jax.experimental import pallas as pl


def kernel(x, index, batch, W1, b1, gn1_w, gn1_b, gn1_ms, W2, b2, gn2_w, gn2_b, gn2_ms, fc_W, fc_b):
    raise NotImplementedError("write your pallas kernel here")


# speedup vs baseline: 9.6916x; 9.6916x over previous
import functools

import jax
import jax.numpy as jnp
from jax import lax
from jax.experimental import pallas as pl
from jax.experimental.pallas import tpu as pltpu
from jax.experimental.pallas import tpu_sc as plsc

N = 10000
E = 320000
G = 64
NB = 10
BLK = 1000
C = 80
NSUB = 16
F32 = jnp.float32

@functools.lru_cache(maxsize=None)
def _mesh():
    return plsc.VectorSubcoreMesh(core_axis_name="c", subcore_axis_name="s")


def _deg_body(dst_hbm, deg_a, deg_b, dst_v, ones_v, zbuf, acc):
    cid = lax.axis_index("c")
    sid = lax.axis_index("s")
    for j in range(C // 16):
        ones_v[pl.ds(j * 16, 16)] = jnp.ones((16,), F32)
    for j in range(640 // 16):
        zbuf[pl.ds(j * 16, 16)] = jnp.zeros((16,), F32)

    @pl.when(sid < 15)
    def _():
        pltpu.sync_copy(zbuf, acc.at[pl.ds(sid * 640, 640)])

    @pl.when(sid == 15)
    def _():
        pltpu.sync_copy(zbuf.at[pl.ds(0, 400)], acc.at[pl.ds(9600, 400)])

    plsc.subcore_barrier()

    def step(k, carry):
        base = cid * (E // 2) + sid * (E // 2 // NSUB) + k * C
        pltpu.sync_copy(dst_hbm.at[pl.ds(base, C)], dst_v)
        pltpu.sync_copy(ones_v, acc.at[dst_v], add=True)
        return carry

    lax.fori_loop(0, E // 2 // NSUB // C, step, 0)
    plsc.subcore_barrier()

    def dump(out_ref, n):
        pltpu.sync_copy(acc.at[pl.ds(sid * 640, n)], zbuf.at[pl.ds(0, n)])
        pltpu.sync_copy(zbuf.at[pl.ds(0, n)], out_ref.at[pl.ds(sid * 640, n)])

    @pl.when(cid == 0)
    def _():
        @pl.when(sid < 15)
        def _():
            dump(deg_a, 640)

        @pl.when(sid == 15)
        def _():
            dump(deg_a, 400)

    @pl.when(cid == 1)
    def _():
        @pl.when(sid < 15)
        def _():
            dump(deg_b, 640)

        @pl.when(sid == 15)
        def _():
            dump(deg_b, 400)


def _deg_call(dst):
    return pl.kernel(
        _deg_body,
        out_type=[jax.ShapeDtypeStruct((N,), F32),
                  jax.ShapeDtypeStruct((N,), F32)],
        mesh=_mesh(),
        scratch_types=[
            pltpu.VMEM((C,), jnp.int32),
            pltpu.VMEM((C,), F32),
            pltpu.VMEM((640,), F32),
            pltpu.VMEM_SHARED((N,), F32),
        ],
    )(dst)


def _make_scat(dh):
    def body(y_hbm, src_hbm, dst_hbm, out_hbm, src_v, dst_v, rows_v, acc, sem):
        cid = lax.axis_index("c")
        sid = lax.axis_index("s")
        yoff = cid * N

        nch = jnp.where(sid == 15, 5, 8)

        def icopy(t, carry):
            r0 = sid * 640 + t * C
            pltpu.sync_copy(y_hbm.at[pl.ds(yoff + r0, C)], rows_v)
            pltpu.sync_copy(rows_v, acc.at[pl.ds(r0, C)])
            return carry

        lax.fori_loop(0, nch, icopy, 0)
        plsc.subcore_barrier()
        off = jnp.zeros((16,), jnp.int32) + yoff

        def step(k, carry):
            base = sid * (E // NSUB) + k * C
            pltpu.sync_copy(src_hbm.at[pl.ds(base, C)], src_v)
            pltpu.sync_copy(dst_hbm.at[pl.ds(base, C)], dst_v)
            for j in range(C // 16):
                src_v[pl.ds(j * 16, 16)] = src_v[pl.ds(j * 16, 16)] + off
            pltpu.async_copy(y_hbm.at[src_v], rows_v, sem).wait()
            pltpu.sync_copy(rows_v, acc.at[dst_v], add=True)
            return carry

        lax.fori_loop(0, E // NSUB // C, step, 0)
        plsc.subcore_barrier()

        def ocopy(t, carry):
            r0 = sid * 640 + t * C
            pltpu.sync_copy(acc.at[pl.ds(r0, C)], rows_v)
            pltpu.sync_copy(rows_v, out_hbm.at[pl.ds(yoff + r0, C)])
            return carry

        lax.fori_loop(0, nch, ocopy, 0)

    def run(y, src, dst):
        return pl.kernel(
            body,
            out_type=jax.ShapeDtypeStruct((2 * N, dh), F32),
            mesh=_mesh(),
            scratch_types=[
                pltpu.VMEM((C,), jnp.int32),
                pltpu.VMEM((C,), jnp.int32),
                pltpu.VMEM((C, dh), F32),
                pltpu.VMEM_SHARED((N, dh), F32),
                pltpu.SemaphoreType.DMA,
            ],
        )(y, src, dst)

    return run


_scat128 = _make_scat(128)


def _scat_edge_body(y_hbm, src_hbm, dst_hbm, out_a, out_b,
                    src_v, dst_v, rows_v, acc, sem):
    cid = lax.axis_index("c")
    sid = lax.axis_index("s")
    nch = jnp.where(sid == 15, 5, 8)

    @pl.when(cid == 1)
    def _():
        def zrow(r, carry):
            for j in range(128 // 16):
                rows_v[r, pl.ds(j * 16, 16)] = jnp.zeros((16,), F32)
            return carry

        lax.fori_loop(0, C, zrow, 0)

    def icopy(t, carry):
        r0 = sid * 640 + t * C

        @pl.when(cid == 0)
        def _():
            pltpu.sync_copy(y_hbm.at[pl.ds(r0, C)], rows_v)

        pltpu.sync_copy(rows_v, acc.at[pl.ds(r0, C)])
        return carry

    lax.fori_loop(0, nch, icopy, 0)
    plsc.subcore_barrier()

    def step(k, carry):
        base = cid * (E // 2) + sid * (E // 2 // NSUB) + k * C
        pltpu.sync_copy(src_hbm.at[pl.ds(base, C)], src_v)
        pltpu.sync_copy(dst_hbm.at[pl.ds(base, C)], dst_v)
        pltpu.async_copy(y_hbm.at[src_v], rows_v, sem).wait()
        pltpu.sync_copy(rows_v, acc.at[dst_v], add=True)
        return carry

    lax.fori_loop(0, E // 2 // NSUB // C, step, 0)
    plsc.subcore_barrier()

    def dump(out_ref):
        def ocopy(t, carry):
            r0 = sid * 640 + t * C
            pltpu.sync_copy(acc.at[pl.ds(r0, C)], rows_v)
            pltpu.sync_copy(rows_v, out_ref.at[pl.ds(r0, C)])
            return carry

        lax.fori_loop(0, nch, ocopy, 0)

    @pl.when(cid == 0)
    def _():
        dump(out_a)

    @pl.when(cid == 1)
    def _():
        dump(out_b)


def _scat_edge(y, src, dst):
    return pl.kernel(
        _scat_edge_body,
        out_type=[jax.ShapeDtypeStruct((N, 128), F32),
                  jax.ShapeDtypeStruct((N, 128), F32)],
        mesh=_mesh(),
        scratch_types=[
            pltpu.VMEM((C,), jnp.int32),
            pltpu.VMEM((C,), jnp.int32),
            pltpu.VMEM((C, 128), F32),
            pltpu.VMEM_SHARED((N, 128), F32),
            pltpu.SemaphoreType.DMA,
        ],
    )(y, src, dst)


def _a1_body(x_ref, w_ref, da_ref, db_ref, y_ref, dinv_ref):
    dinv = lax.rsqrt(da_ref[...] + db_ref[...] + 1.0)
    dinv_ref[...] = dinv
    y_ref[...] = jnp.dot(x_ref[...] * dinv, w_ref[...],
                         preferred_element_type=F32, precision=lax.Precision.HIGHEST)


def _run_a1(x, W1, deg_a, deg_b):
    return pl.pallas_call(
        _a1_body,
        grid=(2, NB),
        in_specs=[
            pl.BlockSpec((BLK, 128), lambda h, i: (i, 0)),
            pl.BlockSpec((128, 128), lambda h, i: (0, h)),
            pl.BlockSpec((BLK, 1), lambda h, i: (i, 0)),
            pl.BlockSpec((BLK, 1), lambda h, i: (i, 0)),
        ],
        out_specs=[
            pl.BlockSpec((BLK, 128), lambda h, i: (h * NB + i, 0)),
            pl.BlockSpec((BLK, 1), lambda h, i: (i, 0)),
        ],
        out_shape=[
            jax.ShapeDtypeStruct((2 * N, 128), F32),
            jax.ShapeDtypeStruct((N, 1), F32),
        ],
    )(x, W1, deg_a, deg_b)


def _onehot(bcol, n_rows):
    iota = lax.broadcasted_iota(jnp.int32, (n_rows, G), 1).astype(F32)
    return (bcol == iota).astype(F32)


def _make_comb_body(hdim, with_cnt, mode):
    def body(s0_ref, s1_ref, dinv_ref, b_ref, bf_ref, h_ref, S1_ref, S2_ref,
             *maybe_S0):
        i = pl.program_id(0)
        if mode == "cat":
            s = jnp.concatenate([s0_ref[...], s1_ref[...]], axis=1)
        else:
            s = s0_ref[...] + s1_ref[...]
        h = s * dinv_ref[...] + b_ref[...]
        h_ref[...] = h
        A = _onehot(bf_ref[...], BLK)
        dn = (((0,), (0,)), ((), ()))
        p1 = lax.dot_general(A, h, dn, preferred_element_type=F32, precision=lax.Precision.HIGHEST)
        p2 = lax.dot_general(A, h * h, dn, preferred_element_type=F32, precision=lax.Precision.HIGHEST)

        @pl.when(i == 0)
        def _():
            S1_ref[...] = jnp.zeros((G, hdim), F32)
            S2_ref[...] = jnp.zeros((G, hdim), F32)
            if with_cnt:
                maybe_S0[0][...] = jnp.zeros((G, 128), F32)

        S1_ref[...] += p1
        S2_ref[...] += p2
        if with_cnt:
            p0 = lax.dot_general(A, jnp.ones((BLK, 128), F32), dn,
                                 preferred_element_type=F32, precision=lax.Precision.HIGHEST)
            maybe_S0[0][...] += p0

    return body


def _run_comb(sa, sb, dinv, bvec, batch_f, hdim, with_cnt, mode):
    if mode == "cat":
        w = hdim // 2
        map_a = lambda i: (i, 0)
        map_b = lambda i: (NB + i, 0)
    else:
        w = hdim
        map_a = lambda i: (i, 0)
        map_b = lambda i: (i, 0)
    out_shape = [
        jax.ShapeDtypeStruct((N, hdim), F32),
        jax.ShapeDtypeStruct((G, hdim), F32),
        jax.ShapeDtypeStruct((G, hdim), F32),
    ]
    out_specs = [
        pl.BlockSpec((BLK, hdim), lambda i: (i, 0)),
        pl.BlockSpec((G, hdim), lambda i: (0, 0)),
        pl.BlockSpec((G, hdim), lambda i: (0, 0)),
    ]
    if with_cnt:
        out_shape.append(jax.ShapeDtypeStruct((G, 128), F32))
        out_specs.append(pl.BlockSpec((G, 128), lambda i: (0, 0)))
    return pl.pallas_call(
        _make_comb_body(hdim, with_cnt, mode),
        grid=(NB,),
        in_specs=[
            pl.BlockSpec((BLK, w), map_a),
            pl.BlockSpec((BLK, w), map_b),
            pl.BlockSpec((BLK, 1), lambda i: (i, 0)),
            pl.BlockSpec((1, hdim), lambda i: (0, 0)),
            pl.BlockSpec((BLK, 1), lambda i: (i, 0)),
        ],
        out_specs=out_specs,
        out_shape=out_shape,
    )(sa, sb, dinv, bvec, batch_f)


def _norm_relu(h, bf, S1, S2, S0, gw, gb, gms):
    cnt = jnp.maximum(S0[:, :1], 1.0)
    mean = S1 / cnt
    var = S2 / cnt + mean * mean * gms * (gms - 2.0)
    istd = lax.rsqrt(var + 1e-5)
    A = _onehot(bf, BLK)
    meanb = jnp.dot(A, gms * mean, preferred_element_type=F32, precision=lax.Precision.HIGHEST)
    istdb = jnp.dot(A, istd, preferred_element_type=F32, precision=lax.Precision.HIGHEST)
    hn = (h - meanb) * istdb * gw + gb
    return jnp.maximum(hn, 0.0)


def _c1_body(h_ref, bf_ref, S1_ref, S2_ref, S0_ref, gw_ref, gb_ref, gms_ref,
             dinv_ref, w_ref, y_ref):
    hr = _norm_relu(h_ref[...], bf_ref[...], S1_ref[...], S2_ref[...],
                    S0_ref[...], gw_ref[...], gb_ref[...], gms_ref[...])
    y_ref[...] = jnp.dot(hr * dinv_ref[...], w_ref[...],
                         preferred_element_type=F32, precision=lax.Precision.HIGHEST)


def _run_c1(h1, batch_f, S1, S2, S0, gw, gb, gms, dinv, W2):
    return pl.pallas_call(
        _c1_body,
        grid=(NB,),
        in_specs=[
            pl.BlockSpec((BLK, 256), lambda i: (i, 0)),
            pl.BlockSpec((BLK, 1), lambda i: (i, 0)),
            pl.BlockSpec((G, 256), lambda i: (0, 0)),
            pl.BlockSpec((G, 256), lambda i: (0, 0)),
            pl.BlockSpec((G, 128), lambda i: (0, 0)),
            pl.BlockSpec((1, 256), lambda i: (0, 0)),
            pl.BlockSpec((1, 256), lambda i: (0, 0)),
            pl.BlockSpec((1, 256), lambda i: (0, 0)),
            pl.BlockSpec((BLK, 1), lambda i: (i, 0)),
            pl.BlockSpec((256, 128), lambda i: (0, 0)),
        ],
        out_specs=pl.BlockSpec((BLK, 128), lambda i: (i, 0)),
        out_shape=jax.ShapeDtypeStruct((N, 128), F32),
    )(h1, batch_f, S1, S2, S0, gw, gb, gms, dinv, W2)


def _f_body(h_ref, bf_ref, S1_ref, S2_ref, S0_ref, gw_ref, gb_ref, gms_ref,
            w_ref, fb_ref, y_ref):
    hr = _norm_relu(h_ref[...], bf_ref[...], S1_ref[...], S2_ref[...],
                    S0_ref[...], gw_ref[...], gb_ref[...], gms_ref[...])
    y_ref[...] = jnp.dot(hr, w_ref[...], preferred_element_type=F32, precision=lax.Precision.HIGHEST) + fb_ref[...]


def _run_f(h2, batch_f, S1, S2, S0, gw, gb, gms, fcw8, fcb8):
    return pl.pallas_call(
        _f_body,
        grid=(NB,),
        in_specs=[
            pl.BlockSpec((BLK, 128), lambda i: (i, 0)),
            pl.BlockSpec((BLK, 1), lambda i: (i, 0)),
            pl.BlockSpec((G, 128), lambda i: (0, 0)),
            pl.BlockSpec((G, 128), lambda i: (0, 0)),
            pl.BlockSpec((G, 128), lambda i: (0, 0)),
            pl.BlockSpec((1, 128), lambda i: (0, 0)),
            pl.BlockSpec((1, 128), lambda i: (0, 0)),
            pl.BlockSpec((1, 128), lambda i: (0, 0)),
            pl.BlockSpec((128, 8), lambda i: (0, 0)),
            pl.BlockSpec((1, 8), lambda i: (0, 0)),
        ],
        out_specs=pl.BlockSpec((BLK, 8), lambda i: (i, 0)),
        out_shape=jax.ShapeDtypeStruct((N, 8), F32),
    )(h2, batch_f, S1, S2, S0, gw, gb, gms, fcw8, fcb8)


def kernel(x, index, batch, W1, b1, gn1_w, gn1_b, gn1_ms, W2, b2,
           gn2_w, gn2_b, gn2_ms, fc_W, fc_b):
    src = index[0]
    dst = index[1]
    batch_f = batch.astype(F32).reshape(N, 1)

    deg_a, deg_b = _deg_call(dst)
    y1, dinv = _run_a1(x, W1, deg_a.reshape(N, 1), deg_b.reshape(N, 1))
    s1 = _scat128(y1, src, dst)
    h1, S1, S2, S0 = _run_comb(s1, s1, dinv, b1.reshape(1, 256), batch_f,
                               256, True, "cat")
    y2 = _run_c1(h1, batch_f, S1, S2, S0, gn1_w.reshape(1, 256),
                 gn1_b.reshape(1, 256), gn1_ms.reshape(1, 256), dinv, W2)
    s2a, s2b = _scat_edge(y2, src, dst)
    h2, T1, T2 = _run_comb(s2a, s2b, dinv, b2.reshape(1, 128), batch_f,
                           128, False, "add")
    fcw8 = jnp.zeros((128, 8), F32).at[:, :2].set(fc_W)
    fcb8 = jnp.zeros((1, 8), F32).at[0, :2].set(fc_b)
    out8 = _run_f(h2, batch_f, T1, T2, S0, gn2_w.reshape(1, 128),
                  gn2_b.reshape(1, 128), gn2_ms.reshape(1, 128), fcw8, fcb8)
    return out8[:, :2]


# speedup vs baseline: 14.1965x; 1.4648x over previous
import functools

import jax
import jax.numpy as jnp
from jax import lax
from jax.experimental import pallas as pl
from jax.experimental.pallas import tpu as pltpu
from jax.experimental.pallas import tpu_sc as plsc

N = 10000
E = 320000
G = 64
NB = 10
BLK = 1000
C = 80
NSUB = 16
F32 = jnp.float32

@functools.lru_cache(maxsize=None)
def _mesh():
    return plsc.VectorSubcoreMesh(core_axis_name="c", subcore_axis_name="s")


def _deg_body(dst_hbm, deg_a, deg_b, dst_v, ones_v, zbuf, acc):
    cid = lax.axis_index("c")
    sid = lax.axis_index("s")
    for j in range(C // 16):
        ones_v[pl.ds(j * 16, 16)] = jnp.ones((16,), F32)
    for j in range(640 // 16):
        zbuf[pl.ds(j * 16, 16)] = jnp.zeros((16,), F32)

    @pl.when(sid < 15)
    def _():
        pltpu.sync_copy(zbuf, acc.at[pl.ds(sid * 640, 640)])

    @pl.when(sid == 15)
    def _():
        pltpu.sync_copy(zbuf.at[pl.ds(0, 400)], acc.at[pl.ds(9600, 400)])

    plsc.subcore_barrier()

    def step(k, carry):
        base = cid * (E // 2) + sid * (E // 2 // NSUB) + k * C
        pltpu.sync_copy(dst_hbm.at[pl.ds(base, C)], dst_v)
        pltpu.sync_copy(ones_v, acc.at[dst_v], add=True)
        return carry

    lax.fori_loop(0, E // 2 // NSUB // C, step, 0)
    plsc.subcore_barrier()

    def dump(out_ref, n):
        pltpu.sync_copy(acc.at[pl.ds(sid * 640, n)], zbuf.at[pl.ds(0, n)])
        pltpu.sync_copy(zbuf.at[pl.ds(0, n)], out_ref.at[pl.ds(sid * 640, n)])

    @pl.when(cid == 0)
    def _():
        @pl.when(sid < 15)
        def _():
            dump(deg_a, 640)

        @pl.when(sid == 15)
        def _():
            dump(deg_a, 400)

    @pl.when(cid == 1)
    def _():
        @pl.when(sid < 15)
        def _():
            dump(deg_b, 640)

        @pl.when(sid == 15)
        def _():
            dump(deg_b, 400)


def _deg_call(dst):
    return pl.kernel(
        _deg_body,
        out_type=[jax.ShapeDtypeStruct((N,), F32),
                  jax.ShapeDtypeStruct((N,), F32)],
        mesh=_mesh(),
        scratch_types=[
            pltpu.VMEM((C,), jnp.int32),
            pltpu.VMEM((C,), F32),
            pltpu.VMEM((640,), F32),
            pltpu.VMEM_SHARED((N,), F32),
        ],
    )(dst)


def _edge_pipeline(y_hbm, src_hbm, dst_hbm, acc, bufs, yoff, ebase, nchunks):
    (src_v, dst_v, rows_v, gsem, ssem) = bufs
    off = jnp.zeros((16,), jnp.int32) + yoff

    def fetch(p, k):
        base = ebase + k * C
        pltpu.sync_copy(src_hbm.at[pl.ds(base, C)], src_v[p])
        pltpu.sync_copy(dst_hbm.at[pl.ds(base, C)], dst_v[p])
        for j in range(C // 16):
            src_v[p][pl.ds(j * 16, 16)] = src_v[p][pl.ds(j * 16, 16)] + off
        pltpu.async_copy(y_hbm.at[src_v[p]], rows_v[p], gsem[p])

    def wait_gather(p):
        pltpu.make_async_copy(y_hbm.at[src_v[p]], rows_v[p], gsem[p]).wait()

    def start_scatter(p):
        pltpu.async_copy(rows_v[p], acc.at[dst_v[p]], ssem[p], add=True)

    def wait_scatter(p):
        pltpu.make_async_copy(rows_v[p], acc.at[dst_v[p]], ssem[p]).wait()

    def step(t, carry):
        @pl.when(t >= 1)
        def _():
            wait_scatter(0)

        fetch(0, 2 * t)

        @pl.when(t >= 1)
        def _():
            wait_gather(1)
            start_scatter(1)

        @pl.when(t >= 1)
        def _():
            wait_scatter(1)

        fetch(1, 2 * t + 1)
        wait_gather(0)
        start_scatter(0)
        return carry

    assert nchunks >= 4
    lax.fori_loop(0, nchunks // 2, step, 0)
    if nchunks % 2:
        wait_scatter(0)
        fetch(0, nchunks - 1)
        wait_gather(1)
        start_scatter(1)
        wait_gather(0)
        start_scatter(0)
        wait_scatter(1)
        wait_scatter(0)
    else:
        wait_gather(1)
        start_scatter(1)
        wait_scatter(0)
        wait_scatter(1)


def _stage_rows(nch, inner):
    def body(t, carry):
        inner(t)
        return carry

    lax.fori_loop(0, nch, body, 0)


def _make_scat(dh):
    def body(y_hbm, src_hbm, dst_hbm, out_hbm,
             src_a, src_b, dst_a, dst_b, rows_a, rows_b, acc,
             gsem_a, gsem_b, ssem_a, ssem_b):
        cid = lax.axis_index("c")
        sid = lax.axis_index("s")
        yoff = cid * N

        nch = jnp.where(sid == 15, 5, 8)

        def icopy(t):
            r0 = sid * 640 + t * C
            pltpu.sync_copy(y_hbm.at[pl.ds(yoff + r0, C)], rows_a)
            pltpu.sync_copy(rows_a, acc.at[pl.ds(r0, C)])

        _stage_rows(nch, icopy)
        plsc.subcore_barrier()

        bufs = ((src_a, src_b), (dst_a, dst_b), (rows_a, rows_b),
                (gsem_a, gsem_b), (ssem_a, ssem_b))
        _edge_pipeline(y_hbm, src_hbm, dst_hbm, acc, bufs, yoff,
                       sid * (E // NSUB), E // NSUB // C)
        plsc.subcore_barrier()

        def ocopy(t):
            r0 = sid * 640 + t * C
            pltpu.sync_copy(acc.at[pl.ds(r0, C)], rows_a)
            pltpu.sync_copy(rows_a, out_hbm.at[pl.ds(yoff + r0, C)])

        _stage_rows(nch, ocopy)

    def run(y, src, dst):
        return pl.kernel(
            body,
            out_type=jax.ShapeDtypeStruct((2 * N, dh), F32),
            mesh=_mesh(),
            scratch_types=[
                pltpu.VMEM((C,), jnp.int32),
                pltpu.VMEM((C,), jnp.int32),
                pltpu.VMEM((C,), jnp.int32),
                pltpu.VMEM((C,), jnp.int32),
                pltpu.VMEM((C, dh), F32),
                pltpu.VMEM((C, dh), F32),
                pltpu.VMEM_SHARED((N, dh), F32),
                pltpu.SemaphoreType.DMA,
                pltpu.SemaphoreType.DMA,
                pltpu.SemaphoreType.DMA,
                pltpu.SemaphoreType.DMA,
            ],
        )(y, src, dst)

    return run


_scat128 = _make_scat(128)


def _scat_edge_body(y_hbm, src_hbm, dst_hbm, out_a, out_b,
                    src_a, src_b, dst_a, dst_b, rows_a, rows_b, acc,
                    gsem_a, gsem_b, ssem_a, ssem_b):
    cid = lax.axis_index("c")
    sid = lax.axis_index("s")
    nch = jnp.where(sid == 15, 5, 8)

    @pl.when(cid == 1)
    def _():
        def zrow(r, carry):
            for j in range(128 // 16):
                rows_a[r, pl.ds(j * 16, 16)] = jnp.zeros((16,), F32)
            return carry

        lax.fori_loop(0, C, zrow, 0)

    def icopy(t):
        r0 = sid * 640 + t * C

        @pl.when(cid == 0)
        def _():
            pltpu.sync_copy(y_hbm.at[pl.ds(r0, C)], rows_a)

        pltpu.sync_copy(rows_a, acc.at[pl.ds(r0, C)])

    _stage_rows(nch, icopy)
    plsc.subcore_barrier()

    bufs = ((src_a, src_b), (dst_a, dst_b), (rows_a, rows_b),
            (gsem_a, gsem_b), (ssem_a, ssem_b))
    _edge_pipeline(y_hbm, src_hbm, dst_hbm, acc, bufs, 0,
                   cid * (E // 2) + sid * (E // 2 // NSUB),
                   E // 2 // NSUB // C)
    plsc.subcore_barrier()

    def dump(out_ref):
        def ocopy(t):
            r0 = sid * 640 + t * C
            pltpu.sync_copy(acc.at[pl.ds(r0, C)], rows_a)
            pltpu.sync_copy(rows_a, out_ref.at[pl.ds(r0, C)])

        _stage_rows(nch, ocopy)

    @pl.when(cid == 0)
    def _():
        dump(out_a)

    @pl.when(cid == 1)
    def _():
        dump(out_b)


def _scat_edge(y, src, dst):
    return pl.kernel(
        _scat_edge_body,
        out_type=[jax.ShapeDtypeStruct((N, 128), F32),
                  jax.ShapeDtypeStruct((N, 128), F32)],
        mesh=_mesh(),
        scratch_types=[
            pltpu.VMEM((C,), jnp.int32),
            pltpu.VMEM((C,), jnp.int32),
            pltpu.VMEM((C,), jnp.int32),
            pltpu.VMEM((C,), jnp.int32),
            pltpu.VMEM((C, 128), F32),
            pltpu.VMEM((C, 128), F32),
            pltpu.VMEM_SHARED((N, 128), F32),
            pltpu.SemaphoreType.DMA,
            pltpu.SemaphoreType.DMA,
            pltpu.SemaphoreType.DMA,
            pltpu.SemaphoreType.DMA,
        ],
    )(y, src, dst)


def _a1_body(x_ref, w_ref, da_ref, db_ref, y_ref, dinv_ref):
    dinv = lax.rsqrt(da_ref[...] + db_ref[...] + 1.0)
    dinv_ref[...] = dinv
    y_ref[...] = jnp.dot(x_ref[...] * dinv, w_ref[...],
                         preferred_element_type=F32, precision=lax.Precision.HIGHEST)


def _run_a1(x, W1, deg_a, deg_b):
    return pl.pallas_call(
        _a1_body,
        grid=(2, NB),
        in_specs=[
            pl.BlockSpec((BLK, 128), lambda h, i: (i, 0)),
            pl.BlockSpec((128, 128), lambda h, i: (0, h)),
            pl.BlockSpec((BLK, 1), lambda h, i: (i, 0)),
            pl.BlockSpec((BLK, 1), lambda h, i: (i, 0)),
        ],
        out_specs=[
            pl.BlockSpec((BLK, 128), lambda h, i: (h * NB + i, 0)),
            pl.BlockSpec((BLK, 1), lambda h, i: (i, 0)),
        ],
        out_shape=[
            jax.ShapeDtypeStruct((2 * N, 128), F32),
            jax.ShapeDtypeStruct((N, 1), F32),
        ],
    )(x, W1, deg_a, deg_b)


def _onehot(bcol, n_rows):
    iota = lax.broadcasted_iota(jnp.int32, (n_rows, G), 1).astype(F32)
    return (bcol == iota).astype(F32)


def _make_comb_body(hdim, with_cnt, mode):
    def body(s0_ref, s1_ref, dinv_ref, b_ref, bf_ref, h_ref, S1_ref, S2_ref,
             *maybe_S0):
        i = pl.program_id(0)
        if mode == "cat":
            s = jnp.concatenate([s0_ref[...], s1_ref[...]], axis=1)
        else:
            s = s0_ref[...] + s1_ref[...]
        h = s * dinv_ref[...] + b_ref[...]
        h_ref[...] = h
        A = _onehot(bf_ref[...], BLK)
        dn = (((0,), (0,)), ((), ()))
        p1 = lax.dot_general(A, h, dn, preferred_element_type=F32, precision=lax.Precision.HIGHEST)
        p2 = lax.dot_general(A, h * h, dn, preferred_element_type=F32, precision=lax.Precision.HIGHEST)

        @pl.when(i == 0)
        def _():
            S1_ref[...] = jnp.zeros((G, hdim), F32)
            S2_ref[...] = jnp.zeros((G, hdim), F32)
            if with_cnt:
                maybe_S0[0][...] = jnp.zeros((G, 128), F32)

        S1_ref[...] += p1
        S2_ref[...] += p2
        if with_cnt:
            p0 = lax.dot_general(A, jnp.ones((BLK, 128), F32), dn,
                                 preferred_element_type=F32, precision=lax.Precision.HIGHEST)
            maybe_S0[0][...] += p0

    return body


def _run_comb(sa, sb, dinv, bvec, batch_f, hdim, with_cnt, mode):
    if mode == "cat":
        w = hdim // 2
        map_a = lambda i: (i, 0)
        map_b = lambda i: (NB + i, 0)
    else:
        w = hdim
        map_a = lambda i: (i, 0)
        map_b = lambda i: (i, 0)
    out_shape = [
        jax.ShapeDtypeStruct((N, hdim), F32),
        jax.ShapeDtypeStruct((G, hdim), F32),
        jax.ShapeDtypeStruct((G, hdim), F32),
    ]
    out_specs = [
        pl.BlockSpec((BLK, hdim), lambda i: (i, 0)),
        pl.BlockSpec((G, hdim), lambda i: (0, 0)),
        pl.BlockSpec((G, hdim), lambda i: (0, 0)),
    ]
    if with_cnt:
        out_shape.append(jax.ShapeDtypeStruct((G, 128), F32))
        out_specs.append(pl.BlockSpec((G, 128), lambda i: (0, 0)))
    return pl.pallas_call(
        _make_comb_body(hdim, with_cnt, mode),
        grid=(NB,),
        in_specs=[
            pl.BlockSpec((BLK, w), map_a),
            pl.BlockSpec((BLK, w), map_b),
            pl.BlockSpec((BLK, 1), lambda i: (i, 0)),
            pl.BlockSpec((1, hdim), lambda i: (0, 0)),
            pl.BlockSpec((BLK, 1), lambda i: (i, 0)),
        ],
        out_specs=out_specs,
        out_shape=out_shape,
    )(sa, sb, dinv, bvec, batch_f)


def _norm_relu(h, bf, S1, S2, S0, gw, gb, gms):
    cnt = jnp.maximum(S0[:, :1], 1.0)
    mean = S1 / cnt
    var = S2 / cnt + mean * mean * gms * (gms - 2.0)
    istd = lax.rsqrt(var + 1e-5)
    A = _onehot(bf, BLK)
    meanb = jnp.dot(A, gms * mean, preferred_element_type=F32, precision=lax.Precision.HIGHEST)
    istdb = jnp.dot(A, istd, preferred_element_type=F32, precision=lax.Precision.HIGHEST)
    hn = (h - meanb) * istdb * gw + gb
    return jnp.maximum(hn, 0.0)


def _c1_body(h_ref, bf_ref, S1_ref, S2_ref, S0_ref, gw_ref, gb_ref, gms_ref,
             dinv_ref, w_ref, y_ref):
    hr = _norm_relu(h_ref[...], bf_ref[...], S1_ref[...], S2_ref[...],
                    S0_ref[...], gw_ref[...], gb_ref[...], gms_ref[...])
    y_ref[...] = jnp.dot(hr * dinv_ref[...], w_ref[...],
                         preferred_element_type=F32, precision=lax.Precision.HIGHEST)


def _run_c1(h1, batch_f, S1, S2, S0, gw, gb, gms, dinv, W2):
    return pl.pallas_call(
        _c1_body,
        grid=(NB,),
        in_specs=[
            pl.BlockSpec((BLK, 256), lambda i: (i, 0)),
            pl.BlockSpec((BLK, 1), lambda i: (i, 0)),
            pl.BlockSpec((G, 256), lambda i: (0, 0)),
            pl.BlockSpec((G, 256), lambda i: (0, 0)),
            pl.BlockSpec((G, 128), lambda i: (0, 0)),
            pl.BlockSpec((1, 256), lambda i: (0, 0)),
            pl.BlockSpec((1, 256), lambda i: (0, 0)),
            pl.BlockSpec((1, 256), lambda i: (0, 0)),
            pl.BlockSpec((BLK, 1), lambda i: (i, 0)),
            pl.BlockSpec((256, 128), lambda i: (0, 0)),
        ],
        out_specs=pl.BlockSpec((BLK, 128), lambda i: (i, 0)),
        out_shape=jax.ShapeDtypeStruct((N, 128), F32),
    )(h1, batch_f, S1, S2, S0, gw, gb, gms, dinv, W2)


def _f_body(h_ref, bf_ref, S1_ref, S2_ref, S0_ref, gw_ref, gb_ref, gms_ref,
            w_ref, fb_ref, y_ref):
    hr = _norm_relu(h_ref[...], bf_ref[...], S1_ref[...], S2_ref[...],
                    S0_ref[...], gw_ref[...], gb_ref[...], gms_ref[...])
    y_ref[...] = jnp.dot(hr, w_ref[...], preferred_element_type=F32, precision=lax.Precision.HIGHEST) + fb_ref[...]


def _run_f(h2, batch_f, S1, S2, S0, gw, gb, gms, fcw8, fcb8):
    return pl.pallas_call(
        _f_body,
        grid=(NB,),
        in_specs=[
            pl.BlockSpec((BLK, 128), lambda i: (i, 0)),
            pl.BlockSpec((BLK, 1), lambda i: (i, 0)),
            pl.BlockSpec((G, 128), lambda i: (0, 0)),
            pl.BlockSpec((G, 128), lambda i: (0, 0)),
            pl.BlockSpec((G, 128), lambda i: (0, 0)),
            pl.BlockSpec((1, 128), lambda i: (0, 0)),
            pl.BlockSpec((1, 128), lambda i: (0, 0)),
            pl.BlockSpec((1, 128), lambda i: (0, 0)),
            pl.BlockSpec((128, 8), lambda i: (0, 0)),
            pl.BlockSpec((1, 8), lambda i: (0, 0)),
        ],
        out_specs=pl.BlockSpec((BLK, 8), lambda i: (i, 0)),
        out_shape=jax.ShapeDtypeStruct((N, 8), F32),
    )(h2, batch_f, S1, S2, S0, gw, gb, gms, fcw8, fcb8)


def kernel(x, index, batch, W1, b1, gn1_w, gn1_b, gn1_ms, W2, b2,
           gn2_w, gn2_b, gn2_ms, fc_W, fc_b):
    src = index[0]
    dst = index[1]
    batch_f = batch.astype(F32).reshape(N, 1)

    deg_a, deg_b = _deg_call(dst)
    y1, dinv = _run_a1(x, W1, deg_a.reshape(N, 1), deg_b.reshape(N, 1))
    s1 = _scat128(y1, src, dst)
    h1, S1, S2, S0 = _run_comb(s1, s1, dinv, b1.reshape(1, 256), batch_f,
                               256, True, "cat")
    y2 = _run_c1(h1, batch_f, S1, S2, S0, gn1_w.reshape(1, 256),
                 gn1_b.reshape(1, 256), gn1_ms.reshape(1, 256), dinv, W2)
    s2a, s2b = _scat_edge(y2, src, dst)
    h2, T1, T2 = _run_comb(s2a, s2b, dinv, b2.reshape(1, 128), batch_f,
                           128, False, "add")
    fcw8 = jnp.zeros((128, 8), F32).at[:, :2].set(fc_W)
    fcb8 = jnp.zeros((1, 8), F32).at[0, :2].set(fc_b)
    out8 = _run_f(h2, batch_f, T1, T2, S0, gn2_w.reshape(1, 128),
                  gn2_b.reshape(1, 128), gn2_ms.reshape(1, 128), fcw8, fcb8)
    return out8[:, :2]


# speedup vs baseline: 16.3489x; 1.1516x over previous
import functools

import jax
import jax.numpy as jnp
from jax import lax
from jax.experimental import pallas as pl
from jax.experimental.pallas import tpu as pltpu
from jax.experimental.pallas import tpu_sc as plsc

N = 10000
E = 320000
G = 64
NB = 10
BLK = 1000
C = 80
NSUB = 16
F32 = jnp.float32

@functools.lru_cache(maxsize=None)
def _mesh():
    return plsc.VectorSubcoreMesh(core_axis_name="c", subcore_axis_name="s")


def _deg_body(dst_hbm, deg_a, deg_b, dst_v, ones_v, zbuf, acc):
    cid = lax.axis_index("c")
    sid = lax.axis_index("s")
    for j in range(C // 16):
        ones_v[pl.ds(j * 16, 16)] = jnp.ones((16,), F32)
    for j in range(640 // 16):
        zbuf[pl.ds(j * 16, 16)] = jnp.zeros((16,), F32)

    @pl.when(sid < 15)
    def _():
        pltpu.sync_copy(zbuf, acc.at[pl.ds(sid * 640, 640)])

    @pl.when(sid == 15)
    def _():
        pltpu.sync_copy(zbuf.at[pl.ds(0, 400)], acc.at[pl.ds(9600, 400)])

    plsc.subcore_barrier()

    def step(k, carry):
        base = cid * (E // 2) + sid * (E // 2 // NSUB) + k * C
        pltpu.sync_copy(dst_hbm.at[pl.ds(base, C)], dst_v)
        pltpu.sync_copy(ones_v, acc.at[dst_v], add=True)
        return carry

    lax.fori_loop(0, E // 2 // NSUB // C, step, 0)
    plsc.subcore_barrier()

    def dump(out_ref, n):
        pltpu.sync_copy(acc.at[pl.ds(sid * 640, n)], zbuf.at[pl.ds(0, n)])
        pltpu.sync_copy(zbuf.at[pl.ds(0, n)], out_ref.at[pl.ds(sid * 640, n)])

    @pl.when(cid == 0)
    def _():
        @pl.when(sid < 15)
        def _():
            dump(deg_a, 640)

        @pl.when(sid == 15)
        def _():
            dump(deg_a, 400)

    @pl.when(cid == 1)
    def _():
        @pl.when(sid < 15)
        def _():
            dump(deg_b, 640)

        @pl.when(sid == 15)
        def _():
            dump(deg_b, 400)


def _deg_call(dst):
    return pl.kernel(
        _deg_body,
        out_type=[jax.ShapeDtypeStruct((N,), F32),
                  jax.ShapeDtypeStruct((N,), F32)],
        mesh=_mesh(),
        scratch_types=[
            pltpu.VMEM((C,), jnp.int32),
            pltpu.VMEM((C,), F32),
            pltpu.VMEM((640,), F32),
            pltpu.VMEM_SHARED((N,), F32),
        ],
    )(dst)


NRING = 4


def _edge_pipeline(y_hbm, src_hbm, dst_hbm, acc, bufs, yoff, ebase, nchunks,
                   do_off):
    (src_v, dst_v, rows_v, gsem, ssem) = bufs
    if do_off:
        off = jnp.zeros((16,), jnp.int32) + yoff

    def fetch(p, k):
        base = ebase + k * C
        pltpu.sync_copy(src_hbm.at[pl.ds(base, C)], src_v[p])
        pltpu.sync_copy(dst_hbm.at[pl.ds(base, C)], dst_v[p])
        if do_off:
            for j in range(C // 16):
                src_v[p][pl.ds(j * 16, 16)] = src_v[p][pl.ds(j * 16, 16)] + off
        pltpu.async_copy(y_hbm.at[src_v[p]], rows_v[p], gsem[p])

    def wait_gather(p):
        pltpu.make_async_copy(y_hbm.at[src_v[p]], rows_v[p], gsem[p]).wait()

    def start_scatter(p):
        pltpu.async_copy(rows_v[p], acc.at[dst_v[p]], ssem[p], add=True)

    def wait_scatter(p):
        pltpu.make_async_copy(rows_v[p], acc.at[dst_v[p]], ssem[p]).wait()

    ngroups, rem = divmod(nchunks, NRING)
    assert ngroups >= 1

    def group(t, carry):
        for p in range(NRING):
            @pl.when(t >= 1)
            def _():
                wait_scatter(p)

            fetch(p, NRING * t + p)
            q = (p - 1) % NRING
            if p == 0:
                @pl.when(t >= 1)
                def _():
                    wait_gather(q)
                    start_scatter(q)
            else:
                wait_gather(q)
                start_scatter(q)
        return carry

    lax.fori_loop(0, ngroups, group, 0)
    for r in range(rem):
        k = ngroups * NRING + r
        wait_scatter(r)
        fetch(r, k)
        q = (r - 1) % NRING
        wait_gather(q)
        start_scatter(q)
    p_last = (nchunks - 1) % NRING
    wait_gather(p_last)
    start_scatter(p_last)
    for p in range(NRING):
        wait_scatter(p)


def _stage_rows(nch, inner):
    def body(t, carry):
        inner(t)
        return carry

    lax.fori_loop(0, nch, body, 0)


def _make_scat(dh):
    def body(y_hbm, src_hbm, dst_hbm, out_hbm, *scr):
        src_v, dst_v, rows_v = scr[0:4], scr[4:8], scr[8:12]
        acc = scr[12]
        gsem, ssem = scr[13:17], scr[17:21]
        rows_a = rows_v[0]
        cid = lax.axis_index("c")
        sid = lax.axis_index("s")
        yoff = cid * N

        nch = jnp.where(sid == 15, 5, 8)

        def icopy(t):
            r0 = sid * 640 + t * C
            pltpu.sync_copy(y_hbm.at[pl.ds(yoff + r0, C)], rows_a)
            pltpu.sync_copy(rows_a, acc.at[pl.ds(r0, C)])

        _stage_rows(nch, icopy)
        plsc.subcore_barrier()

        bufs = (src_v, dst_v, rows_v, gsem, ssem)
        _edge_pipeline(y_hbm, src_hbm, dst_hbm, acc, bufs, yoff,
                       sid * (E // NSUB), E // NSUB // C, True)
        plsc.subcore_barrier()

        def ocopy(t):
            r0 = sid * 640 + t * C
            pltpu.sync_copy(acc.at[pl.ds(r0, C)], rows_a)
            pltpu.sync_copy(rows_a, out_hbm.at[pl.ds(yoff + r0, C)])

        _stage_rows(nch, ocopy)

    def run(y, src, dst):
        return pl.kernel(
            body,
            out_type=jax.ShapeDtypeStruct((2 * N, dh), F32),
            mesh=_mesh(),
            scratch_types=(
                [pltpu.VMEM((C,), jnp.int32)] * (2 * NRING)
                + [pltpu.VMEM((C, dh), F32)] * NRING
                + [pltpu.VMEM_SHARED((N, dh), F32)]
                + [pltpu.SemaphoreType.DMA] * (2 * NRING)
            ),
        )(y, src, dst)

    return run


_scat128 = _make_scat(128)


def _scat_edge_body(y_hbm, src_hbm, dst_hbm, out_a, out_b, *scr):
    src_v, dst_v, rows_v = scr[0:4], scr[4:8], scr[8:12]
    acc = scr[12]
    gsem, ssem = scr[13:17], scr[17:21]
    rows_a = rows_v[0]
    cid = lax.axis_index("c")
    sid = lax.axis_index("s")
    nch = jnp.where(sid == 15, 5, 8)

    @pl.when(cid == 1)
    def _():
        def zrow(r, carry):
            for j in range(128 // 16):
                rows_a[r, pl.ds(j * 16, 16)] = jnp.zeros((16,), F32)
            return carry

        lax.fori_loop(0, C, zrow, 0)

    def icopy(t):
        r0 = sid * 640 + t * C

        @pl.when(cid == 0)
        def _():
            pltpu.sync_copy(y_hbm.at[pl.ds(r0, C)], rows_a)

        pltpu.sync_copy(rows_a, acc.at[pl.ds(r0, C)])

    _stage_rows(nch, icopy)
    plsc.subcore_barrier()

    bufs = (src_v, dst_v, rows_v, gsem, ssem)
    _edge_pipeline(y_hbm, src_hbm, dst_hbm, acc, bufs, 0,
                   cid * (E // 2) + sid * (E // 2 // NSUB),
                   E // 2 // NSUB // C, False)
    plsc.subcore_barrier()

    def dump(out_ref):
        def ocopy(t):
            r0 = sid * 640 + t * C
            pltpu.sync_copy(acc.at[pl.ds(r0, C)], rows_a)
            pltpu.sync_copy(rows_a, out_ref.at[pl.ds(r0, C)])

        _stage_rows(nch, ocopy)

    @pl.when(cid == 0)
    def _():
        dump(out_a)

    @pl.when(cid == 1)
    def _():
        dump(out_b)


def _scat_edge(y, src, dst):
    return pl.kernel(
        _scat_edge_body,
        out_type=[jax.ShapeDtypeStruct((N, 128), F32),
                  jax.ShapeDtypeStruct((N, 128), F32)],
        mesh=_mesh(),
        scratch_types=(
            [pltpu.VMEM((C,), jnp.int32)] * (2 * NRING)
            + [pltpu.VMEM((C, 128), F32)] * NRING
            + [pltpu.VMEM_SHARED((N, 128), F32)]
            + [pltpu.SemaphoreType.DMA] * (2 * NRING)
        ),
    )(y, src, dst)


def _a1_body(x_ref, w_ref, da_ref, db_ref, y_ref, dinv_ref):
    dinv = lax.rsqrt(da_ref[...] + db_ref[...] + 1.0)
    dinv_ref[...] = dinv
    y_ref[...] = jnp.dot(x_ref[...] * dinv, w_ref[...],
                         preferred_element_type=F32, precision=lax.Precision.HIGHEST)


def _run_a1(x, W1, deg_a, deg_b):
    return pl.pallas_call(
        _a1_body,
        grid=(2, NB),
        in_specs=[
            pl.BlockSpec((BLK, 128), lambda h, i: (i, 0)),
            pl.BlockSpec((128, 128), lambda h, i: (0, h)),
            pl.BlockSpec((BLK, 1), lambda h, i: (i, 0)),
            pl.BlockSpec((BLK, 1), lambda h, i: (i, 0)),
        ],
        out_specs=[
            pl.BlockSpec((BLK, 128), lambda h, i: (h * NB + i, 0)),
            pl.BlockSpec((BLK, 1), lambda h, i: (i, 0)),
        ],
        out_shape=[
            jax.ShapeDtypeStruct((2 * N, 128), F32),
            jax.ShapeDtypeStruct((N, 1), F32),
        ],
    )(x, W1, deg_a, deg_b)


def _onehot(bcol, n_rows):
    iota = lax.broadcasted_iota(jnp.int32, (n_rows, G), 1).astype(F32)
    return (bcol == iota).astype(F32)


def _make_comb_body(hdim, with_cnt, mode):
    def body(s0_ref, s1_ref, dinv_ref, b_ref, bf_ref, h_ref, S1_ref, S2_ref,
             *maybe_S0):
        i = pl.program_id(0)
        if mode == "cat":
            s = jnp.concatenate([s0_ref[...], s1_ref[...]], axis=1)
        else:
            s = s0_ref[...] + s1_ref[...]
        h = s * dinv_ref[...] + b_ref[...]
        h_ref[...] = h
        A = _onehot(bf_ref[...], BLK)
        dn = (((0,), (0,)), ((), ()))
        p1 = lax.dot_general(A, h, dn, preferred_element_type=F32, precision=lax.Precision.HIGHEST)
        p2 = lax.dot_general(A, h * h, dn, preferred_element_type=F32, precision=lax.Precision.HIGHEST)

        @pl.when(i == 0)
        def _():
            S1_ref[...] = jnp.zeros((G, hdim), F32)
            S2_ref[...] = jnp.zeros((G, hdim), F32)
            if with_cnt:
                maybe_S0[0][...] = jnp.zeros((G, 128), F32)

        S1_ref[...] += p1
        S2_ref[...] += p2
        if with_cnt:
            p0 = lax.dot_general(A, jnp.ones((BLK, 128), F32), dn,
                                 preferred_element_type=F32, precision=lax.Precision.HIGHEST)
            maybe_S0[0][...] += p0

    return body


def _run_comb(sa, sb, dinv, bvec, batch_f, hdim, with_cnt, mode):
    if mode == "cat":
        w = hdim // 2
        map_a = lambda i: (i, 0)
        map_b = lambda i: (NB + i, 0)
    else:
        w = hdim
        map_a = lambda i: (i, 0)
        map_b = lambda i: (i, 0)
    out_shape = [
        jax.ShapeDtypeStruct((N, hdim), F32),
        jax.ShapeDtypeStruct((G, hdim), F32),
        jax.ShapeDtypeStruct((G, hdim), F32),
    ]
    out_specs = [
        pl.BlockSpec((BLK, hdim), lambda i: (i, 0)),
        pl.BlockSpec((G, hdim), lambda i: (0, 0)),
        pl.BlockSpec((G, hdim), lambda i: (0, 0)),
    ]
    if with_cnt:
        out_shape.append(jax.ShapeDtypeStruct((G, 128), F32))
        out_specs.append(pl.BlockSpec((G, 128), lambda i: (0, 0)))
    return pl.pallas_call(
        _make_comb_body(hdim, with_cnt, mode),
        grid=(NB,),
        in_specs=[
            pl.BlockSpec((BLK, w), map_a),
            pl.BlockSpec((BLK, w), map_b),
            pl.BlockSpec((BLK, 1), lambda i: (i, 0)),
            pl.BlockSpec((1, hdim), lambda i: (0, 0)),
            pl.BlockSpec((BLK, 1), lambda i: (i, 0)),
        ],
        out_specs=out_specs,
        out_shape=out_shape,
    )(sa, sb, dinv, bvec, batch_f)


def _norm_relu(h, bf, S1, S2, S0, gw, gb, gms):
    cnt = jnp.maximum(S0[:, :1], 1.0)
    mean = S1 / cnt
    var = S2 / cnt + mean * mean * gms * (gms - 2.0)
    istd = lax.rsqrt(var + 1e-5)
    A = _onehot(bf, BLK)
    meanb = jnp.dot(A, gms * mean, preferred_element_type=F32, precision=lax.Precision.HIGHEST)
    istdb = jnp.dot(A, istd, preferred_element_type=F32, precision=lax.Precision.HIGHEST)
    hn = (h - meanb) * istdb * gw + gb
    return jnp.maximum(hn, 0.0)


def _c1_body(h_ref, bf_ref, S1_ref, S2_ref, S0_ref, gw_ref, gb_ref, gms_ref,
             dinv_ref, w_ref, y_ref):
    hr = _norm_relu(h_ref[...], bf_ref[...], S1_ref[...], S2_ref[...],
                    S0_ref[...], gw_ref[...], gb_ref[...], gms_ref[...])
    y_ref[...] = jnp.dot(hr * dinv_ref[...], w_ref[...],
                         preferred_element_type=F32, precision=lax.Precision.HIGHEST)


def _run_c1(h1, batch_f, S1, S2, S0, gw, gb, gms, dinv, W2):
    return pl.pallas_call(
        _c1_body,
        grid=(NB,),
        in_specs=[
            pl.BlockSpec((BLK, 256), lambda i: (i, 0)),
            pl.BlockSpec((BLK, 1), lambda i: (i, 0)),
            pl.BlockSpec((G, 256), lambda i: (0, 0)),
            pl.BlockSpec((G, 256), lambda i: (0, 0)),
            pl.BlockSpec((G, 128), lambda i: (0, 0)),
            pl.BlockSpec((1, 256), lambda i: (0, 0)),
            pl.BlockSpec((1, 256), lambda i: (0, 0)),
            pl.BlockSpec((1, 256), lambda i: (0, 0)),
            pl.BlockSpec((BLK, 1), lambda i: (i, 0)),
            pl.BlockSpec((256, 128), lambda i: (0, 0)),
        ],
        out_specs=pl.BlockSpec((BLK, 128), lambda i: (i, 0)),
        out_shape=jax.ShapeDtypeStruct((N, 128), F32),
    )(h1, batch_f, S1, S2, S0, gw, gb, gms, dinv, W2)


def _f_body(h_ref, bf_ref, S1_ref, S2_ref, S0_ref, gw_ref, gb_ref, gms_ref,
            w_ref, fb_ref, y_ref):
    hr = _norm_relu(h_ref[...], bf_ref[...], S1_ref[...], S2_ref[...],
                    S0_ref[...], gw_ref[...], gb_ref[...], gms_ref[...])
    y_ref[...] = jnp.dot(hr, w_ref[...], preferred_element_type=F32, precision=lax.Precision.HIGHEST) + fb_ref[...]


def _run_f(h2, batch_f, S1, S2, S0, gw, gb, gms, fcw8, fcb8):
    return pl.pallas_call(
        _f_body,
        grid=(NB,),
        in_specs=[
            pl.BlockSpec((BLK, 128), lambda i: (i, 0)),
            pl.BlockSpec((BLK, 1), lambda i: (i, 0)),
            pl.BlockSpec((G, 128), lambda i: (0, 0)),
            pl.BlockSpec((G, 128), lambda i: (0, 0)),
            pl.BlockSpec((G, 128), lambda i: (0, 0)),
            pl.BlockSpec((1, 128), lambda i: (0, 0)),
            pl.BlockSpec((1, 128), lambda i: (0, 0)),
            pl.BlockSpec((1, 128), lambda i: (0, 0)),
            pl.BlockSpec((128, 8), lambda i: (0, 0)),
            pl.BlockSpec((1, 8), lambda i: (0, 0)),
        ],
        out_specs=pl.BlockSpec((BLK, 8), lambda i: (i, 0)),
        out_shape=jax.ShapeDtypeStruct((N, 8), F32),
    )(h2, batch_f, S1, S2, S0, gw, gb, gms, fcw8, fcb8)


def kernel(x, index, batch, W1, b1, gn1_w, gn1_b, gn1_ms, W2, b2,
           gn2_w, gn2_b, gn2_ms, fc_W, fc_b):
    src = index[0]
    dst = index[1]
    batch_f = batch.astype(F32).reshape(N, 1)

    deg_a, deg_b = _deg_call(dst)
    y1, dinv = _run_a1(x, W1, deg_a.reshape(N, 1), deg_b.reshape(N, 1))
    s1 = _scat128(y1, src, dst)
    h1, S1, S2, S0 = _run_comb(s1, s1, dinv, b1.reshape(1, 256), batch_f,
                               256, True, "cat")
    y2 = _run_c1(h1, batch_f, S1, S2, S0, gn1_w.reshape(1, 256),
                 gn1_b.reshape(1, 256), gn1_ms.reshape(1, 256), dinv, W2)
    s2a, s2b = _scat_edge(y2, src, dst)
    h2, T1, T2 = _run_comb(s2a, s2b, dinv, b2.reshape(1, 128), batch_f,
                           128, False, "add")
    fcw8 = jnp.zeros((128, 8), F32).at[:, :2].set(fc_W)
    fcb8 = jnp.zeros((1, 8), F32).at[0, :2].set(fc_b)
    out8 = _run_f(h2, batch_f, T1, T2, S0, gn2_w.reshape(1, 128),
                  gn2_b.reshape(1, 128), gn2_ms.reshape(1, 128), fcw8, fcb8)
    return out8[:, :2]
